# baseline (speedup 1.0000x reference)
import functools

import jax
import jax.numpy as jnp
import numpy as np
from jax import lax
from jax.experimental import pallas as pl
from jax.experimental.pallas import tpu as pltpu
from jax.experimental.pallas import tpu_sc as plsc

_N = 512
_E = 16384
_U = _N * (_N - 1) // 2
_NUM_LOGITS = _U + _N
_NW = 32
_EPW = _E // _NW
_CHUNK = 128
_SLICE = _N * _N // 16
_GPW = _N * _N // _NW
_TILE = 16384
_DPW = _NUM_LOGITS // 16


def _sc_mesh():
    return plsc.VectorSubcoreMesh(core_axis_name="c", subcore_axis_name="s")


def _edge_counts(edge_index):
    @functools.partial(
        pl.kernel,
        out_type=jax.ShapeDtypeStruct((2, _N, _N), jnp.float32),
        mesh=_sc_mesh(),
        scratch_types=[
            pltpu.VMEM((_EPW,), jnp.int32),
            pltpu.VMEM((_EPW,), jnp.int32),
            pltpu.VMEM((_EPW // _CHUNK, _CHUNK), jnp.int32),
            pltpu.VMEM((_CHUNK,), jnp.float32),
            pltpu.VMEM((2048,), jnp.float32),
            pltpu.VMEM((_SLICE // _N, _N), jnp.float32),
            pltpu.VMEM_SHARED((_N * _N,), jnp.float32),
            pltpu.SemaphoreType.DMA,
        ],
    )
    def k(edges, out, src_v, dst_v, idx_v, ones_v, zeros_v, rows_v, acc, sem):
        cid = lax.axis_index("c")
        sid = lax.axis_index("s")
        wid = cid * 16 + sid
        base = wid * _EPW
        pltpu.sync_copy(edges.at[0, pl.ds(base, _EPW)], src_v)
        pltpu.sync_copy(edges.at[1, pl.ds(base, _EPW)], dst_v)

        def _zero(i, carry):
            zeros_v[pl.ds(i * 16, 16)] = jnp.zeros((16,), jnp.float32)
            return carry

        lax.fori_loop(0, 128, _zero, 0)
        for t in range(_CHUNK // 16):
            ones_v[pl.ds(t * 16, 16)] = jnp.ones((16,), jnp.float32)
        for t in range(_EPW // 16):
            s = src_v[pl.ds(t * 16, 16)]
            d = dst_v[pl.ds(t * 16, 16)]
            idx_v[t * 16 // _CHUNK, pl.ds(t * 16 % _CHUNK, 16)] = d * _N + s
        for q in range(_SLICE // 2048):
            pltpu.sync_copy(zeros_v, acc.at[pl.ds(sid * _SLICE + q * 2048, 2048)])
        plsc.subcore_barrier()
        for c in range(_EPW // _CHUNK):
            pltpu.sync_copy(ones_v, acc.at[idx_v.at[c]], add=True)
        plsc.subcore_barrier()
        nrow = _SLICE // _N
        stages = [
            pltpu.async_copy(acc.at[pl.ds(sid * _SLICE + r * _N, _N)],
                             rows_v.at[r], sem)
            for r in range(nrow)
        ]
        for s_ in stages:
            s_.wait()
        pltpu.sync_copy(rows_v, out.at[cid, pl.ds(sid * nrow, nrow)])

    return k(edge_index)


def _encode_decode(counts, x, W1, b1, g1, be1, W2, b2, g2, be2,
                   Wmu, bmu, Wlv, blv, Wd1, bd1, eps, Wd2, bd2):
    def body(c_ref, x_ref, w1, b1r, g1r, be1r, w2, b2r, g2r, be2r,
             wmu, bmur, wlv, blvr, wd1, bd1r, eps_ref,
             wq0, wq1, wq2, wq3, b_ref, o_ref, a_scr):
        @pl.when(pl.program_id(0) == 0)
        def _():
            A = c_ref[0] + c_ref[1]
            deg = jnp.sum(A, axis=1, keepdims=True) + 1.0
            dinv = lax.rsqrt(deg)

            def gcn_bn_relu(h, W, b, g, be):
                gmat = jnp.dot(h, W, preferred_element_type=jnp.float32) * dinv
                m = (jnp.dot(A, gmat, preferred_element_type=jnp.float32)
                     + gmat) * dinv + b
                mu_ = jnp.sum(m, axis=0, keepdims=True) * (1.0 / _N)
                var = jnp.sum((m - mu_) ** 2, axis=0, keepdims=True) * (1.0 / _N)
                return jnp.maximum(
                    (m - mu_) * lax.rsqrt(var + 1e-5) * g + be, 0.0)

            h = gcn_bn_relu(x_ref[...], w1[...], b1r[...], g1r[...], be1r[...])
            h = gcn_bn_relu(h, w2[...], b2r[...], g2r[...], be2r[...])
            hg = jnp.sum(h, axis=0, keepdims=True) * (1.0 / _N)
            mu = (jnp.dot(hg, wmu[...], preferred_element_type=jnp.float32)
                  + bmur[...])
            logvar = (jnp.dot(hg, wlv[...], preferred_element_type=jnp.float32)
                      + blvr[...])
            z = mu + eps_ref[...] * jnp.exp(0.5 * logvar)
            a_scr[...] = jnp.maximum(
                jnp.dot(z, wd1[...], preferred_element_type=jnp.float32)
                + bd1r[...], 0.0)

        acc = jnp.broadcast_to(b_ref[...], (1, _TILE))
        for q, w in enumerate((wq0, wq1, wq2, wq3)):
            acc = acc + jnp.dot(a_scr[:, q * 64:(q + 1) * 64], w[...],
                                preferred_element_type=jnp.float32)
        o_ref[...] = jnp.tanh(acc).reshape(_TILE)

    enc_spec = lambda *shape: pl.BlockSpec(shape, lambda g: (0,) * len(shape))
    return pl.pallas_call(
        body,
        grid=(pl.cdiv(_NUM_LOGITS, _TILE),),
        in_specs=[
            enc_spec(2, _N, _N),
            enc_spec(*x.shape),
            enc_spec(*W1.shape), enc_spec(*b1.shape),
            enc_spec(*g1.shape), enc_spec(*be1.shape),
            enc_spec(*W2.shape), enc_spec(*b2.shape),
            enc_spec(*g2.shape), enc_spec(*be2.shape),
            enc_spec(*Wmu.shape), enc_spec(*bmu.shape),
            enc_spec(*Wlv.shape), enc_spec(*blv.shape),
            enc_spec(*Wd1.shape), enc_spec(*bd1.shape),
            enc_spec(*eps.shape),
            pl.BlockSpec((64, _TILE), lambda g: (0, g)),
            pl.BlockSpec((64, _TILE), lambda g: (1, g)),
            pl.BlockSpec((64, _TILE), lambda g: (2, g)),
            pl.BlockSpec((64, _TILE), lambda g: (3, g)),
            pl.BlockSpec((_TILE,), lambda g: (g,)),
        ],
        out_specs=pl.BlockSpec((_TILE,), lambda g: (g,)),
        out_shape=jax.ShapeDtypeStruct((_NUM_LOGITS,), jnp.float32),
        scratch_shapes=[pltpu.VMEM((1, 256), jnp.float32)],
        compiler_params=pltpu.CompilerParams(
            dimension_semantics=("arbitrary",)),
    )(counts, x, W1, b1, g1, be1, W2, b2, g2, be2,
      Wmu, bmu, Wlv, blv, Wd1, bd1, eps, Wd2, Wd2, Wd2, Wd2, bd2)


def _adj_gather(d, idx):
    @functools.partial(
        pl.kernel,
        out_type=jax.ShapeDtypeStruct((_N, _N), jnp.float32),
        mesh=_sc_mesh(),
        scratch_types=[
            pltpu.VMEM((_GPW // _CHUNK, _CHUNK), jnp.int32),
            pltpu.VMEM((_N // _NW, _N), jnp.float32),
            pltpu.VMEM((_DPW,), jnp.float32),
            pltpu.VMEM_SHARED((_NUM_LOGITS,), jnp.float32),
            pltpu.SemaphoreType.DMA,
        ],
    )
    def k(d_hbm, idx_hbm, out, idx_v, rows_v, stage_v, d_sh, sem):
        cid = lax.axis_index("c")
        sid = lax.axis_index("s")
        wid = cid * 16 + sid
        pltpu.sync_copy(d_hbm.at[pl.ds(sid * _DPW, _DPW)], stage_v)
        pltpu.sync_copy(stage_v, d_sh.at[pl.ds(sid * _DPW, _DPW)])
        pltpu.sync_copy(idx_hbm.at[wid], idx_v)
        plsc.subcore_barrier()
        nch = _GPW // _CHUNK
        cpr = _N // _CHUNK
        copies = [
            pltpu.async_copy(d_sh.at[idx_v.at[j]],
                             rows_v.at[j // cpr, pl.ds((j % cpr) * _CHUNK,
                                                       _CHUNK)], sem)
            for j in range(nch)
        ]
        for c_ in copies:
            c_.wait()
        pltpu.sync_copy(rows_v, out.at[pl.ds(wid * (_N // _NW), _N // _NW)])

    return k(d, idx)


def _adj_index_map():
    ii = np.arange(_N, dtype=np.int32)[:, None]
    jj = np.arange(_N, dtype=np.int32)[None, :]
    lo = np.minimum(ii, jj)
    hi = np.maximum(ii, jj)
    start = lo * (_N - 1) - (lo * (lo - 1)) // 2
    k = start + hi - lo - 1
    k = np.where(ii == jj, _U + ii, k).astype(np.int32)
    return jnp.asarray(k.reshape(_NW, _GPW // _CHUNK, _CHUNK))


def kernel(x, edge_index, eps, W1, b1, g1, be1, W2, b2, g2, be2,
           Wmu, bmu, Wlv, blv, Wd1, bd1, Wd2, bd2):
    counts = _edge_counts(edge_index)
    d = _encode_decode(
        counts, x, W1, b1.reshape(1, -1), g1.reshape(1, -1), be1.reshape(1, -1),
        W2, b2.reshape(1, -1), g2.reshape(1, -1), be2.reshape(1, -1),
        Wmu, bmu.reshape(1, -1), Wlv, blv.reshape(1, -1),
        Wd1, bd1.reshape(1, -1), eps, Wd2, bd2)
    return _adj_gather(d, _adj_index_map())


# speedup vs baseline: 1.0379x; 1.0379x over previous
import functools

import jax
import jax.numpy as jnp
import numpy as np
from jax import lax
from jax.experimental import pallas as pl
from jax.experimental.pallas import tpu as pltpu
from jax.experimental.pallas import tpu_sc as plsc

_N = 512
_E = 16384
_U = _N * (_N - 1) // 2
_NUM_LOGITS = _U + _N
_NW = 32
_EPW = _E // _NW
_CHUNK = 128
_SLICE = _N * _N // 16
_GPW = _N * _N // _NW
_TILE = 12288
_DPW = _NUM_LOGITS // 16


def _sc_mesh():
    return plsc.VectorSubcoreMesh(core_axis_name="c", subcore_axis_name="s")


def _edge_counts(edge_index):
    @functools.partial(
        pl.kernel,
        out_type=jax.ShapeDtypeStruct((2, _N, _N), jnp.float32),
        mesh=_sc_mesh(),
        scratch_types=[
            pltpu.VMEM((_EPW,), jnp.int32),
            pltpu.VMEM((_EPW,), jnp.int32),
            pltpu.VMEM((_EPW // _CHUNK, _CHUNK), jnp.int32),
            pltpu.VMEM((_CHUNK,), jnp.float32),
            pltpu.VMEM((2048,), jnp.float32),
            pltpu.VMEM((_SLICE // _N, _N), jnp.float32),
            pltpu.VMEM_SHARED((_N * _N,), jnp.float32),
            pltpu.SemaphoreType.DMA,
        ],
    )
    def k(edges, out, src_v, dst_v, idx_v, ones_v, zeros_v, rows_v, acc, sem):
        cid = lax.axis_index("c")
        sid = lax.axis_index("s")
        wid = cid * 16 + sid
        base = wid * _EPW
        pltpu.sync_copy(edges.at[0, pl.ds(base, _EPW)], src_v)
        pltpu.sync_copy(edges.at[1, pl.ds(base, _EPW)], dst_v)

        def _zero(i, carry):
            zeros_v[pl.ds(i * 16, 16)] = jnp.zeros((16,), jnp.float32)
            return carry

        lax.fori_loop(0, 128, _zero, 0)
        for t in range(_CHUNK // 16):
            ones_v[pl.ds(t * 16, 16)] = jnp.ones((16,), jnp.float32)
        for t in range(_EPW // 16):
            s = src_v[pl.ds(t * 16, 16)]
            d = dst_v[pl.ds(t * 16, 16)]
            idx_v[t * 16 // _CHUNK, pl.ds(t * 16 % _CHUNK, 16)] = d * _N + s
        for q in range(_SLICE // 2048):
            pltpu.sync_copy(zeros_v, acc.at[pl.ds(sid * _SLICE + q * 2048, 2048)])
        plsc.subcore_barrier()
        for c in range(_EPW // _CHUNK):
            pltpu.sync_copy(ones_v, acc.at[idx_v.at[c]], add=True)
        plsc.subcore_barrier()
        nrow = _SLICE // _N
        stages = [
            pltpu.async_copy(acc.at[pl.ds(sid * _SLICE + r * _N, _N)],
                             rows_v.at[r], sem)
            for r in range(nrow)
        ]
        for s_ in stages:
            s_.wait()
        pltpu.sync_copy(rows_v, out.at[cid, pl.ds(sid * nrow, nrow)])

    return k(edge_index)


def _encode_decode(counts, x, W1, b1, g1, be1, W2, b2, g2, be2,
                   Wmu, bmu, Wlv, blv, Wd1, bd1, eps, Wd2, bd2):
    def body(c_ref, x_ref, w1, b1r, g1r, be1r, w2, b2r, g2r, be2r,
             wmu, bmur, wlv, blvr, wd1, bd1r, eps_ref,
             wq0, wq1, wq2, wq3, b_ref, o_ref, a_scr):
        @pl.when(pl.program_id(0) == 0)
        def _():
            A = c_ref[0] + c_ref[1]
            deg = jnp.sum(A, axis=1, keepdims=True) + 1.0
            dinv = lax.rsqrt(deg)

            def gcn_bn_relu(h, W, b, g, be):
                gmat = jnp.dot(h, W, preferred_element_type=jnp.float32) * dinv
                m = (jnp.dot(A, gmat, preferred_element_type=jnp.float32)
                     + gmat) * dinv + b
                mu_ = jnp.sum(m, axis=0, keepdims=True) * (1.0 / _N)
                var = jnp.sum((m - mu_) ** 2, axis=0, keepdims=True) * (1.0 / _N)
                return jnp.maximum(
                    (m - mu_) * lax.rsqrt(var + 1e-5) * g + be, 0.0)

            h = gcn_bn_relu(x_ref[...], w1[...], b1r[...], g1r[...], be1r[...])
            h = gcn_bn_relu(h, w2[...], b2r[...], g2r[...], be2r[...])
            hg = jnp.sum(h, axis=0, keepdims=True) * (1.0 / _N)
            mu = (jnp.dot(hg, wmu[...], preferred_element_type=jnp.float32)
                  + bmur[...])
            logvar = (jnp.dot(hg, wlv[...], preferred_element_type=jnp.float32)
                      + blvr[...])
            z = mu + eps_ref[...] * jnp.exp(0.5 * logvar)
            a_scr[...] = jnp.maximum(
                jnp.dot(z, wd1[...], preferred_element_type=jnp.float32)
                + bd1r[...], 0.0)

        acc = jnp.broadcast_to(b_ref[...], (1, _TILE))
        for q, w in enumerate((wq0, wq1, wq2, wq3)):
            acc = acc + jnp.dot(a_scr[:, q * 64:(q + 1) * 64], w[...],
                                preferred_element_type=jnp.float32)
        o_ref[...] = jnp.tanh(acc).reshape(_TILE)

    enc_spec = lambda *shape: pl.BlockSpec(shape, lambda g: (0,) * len(shape))
    return pl.pallas_call(
        body,
        grid=(pl.cdiv(_NUM_LOGITS, _TILE),),
        in_specs=[
            enc_spec(2, _N, _N),
            enc_spec(*x.shape),
            enc_spec(*W1.shape), enc_spec(*b1.shape),
            enc_spec(*g1.shape), enc_spec(*be1.shape),
            enc_spec(*W2.shape), enc_spec(*b2.shape),
            enc_spec(*g2.shape), enc_spec(*be2.shape),
            enc_spec(*Wmu.shape), enc_spec(*bmu.shape),
            enc_spec(*Wlv.shape), enc_spec(*blv.shape),
            enc_spec(*Wd1.shape), enc_spec(*bd1.shape),
            enc_spec(*eps.shape),
            pl.BlockSpec((64, _TILE), lambda g: (0, g)),
            pl.BlockSpec((64, _TILE), lambda g: (1, g)),
            pl.BlockSpec((64, _TILE), lambda g: (2, g)),
            pl.BlockSpec((64, _TILE), lambda g: (3, g)),
            pl.BlockSpec((_TILE,), lambda g: (g,)),
        ],
        out_specs=pl.BlockSpec((_TILE,), lambda g: (g,)),
        out_shape=jax.ShapeDtypeStruct((_NUM_LOGITS,), jnp.float32),
        scratch_shapes=[pltpu.VMEM((1, 256), jnp.float32)],
        compiler_params=pltpu.CompilerParams(
            dimension_semantics=("arbitrary",)),
    )(counts, x, W1, b1, g1, be1, W2, b2, g2, be2,
      Wmu, bmu, Wlv, blv, Wd1, bd1, eps, Wd2, Wd2, Wd2, Wd2, bd2)


def _adj_gather(d, idx):
    @functools.partial(
        pl.kernel,
        out_type=jax.ShapeDtypeStruct((_N, _N), jnp.float32),
        mesh=_sc_mesh(),
        scratch_types=[
            pltpu.VMEM((_GPW // _CHUNK, _CHUNK), jnp.int32),
            pltpu.VMEM((_N // _NW, _N), jnp.float32),
            pltpu.VMEM((_DPW,), jnp.float32),
            pltpu.VMEM_SHARED((_NUM_LOGITS,), jnp.float32),
            pltpu.SemaphoreType.DMA,
        ],
    )
    def k(d_hbm, idx_hbm, out, idx_v, rows_v, stage_v, d_sh, sem):
        cid = lax.axis_index("c")
        sid = lax.axis_index("s")
        wid = cid * 16 + sid
        pltpu.sync_copy(d_hbm.at[pl.ds(sid * _DPW, _DPW)], stage_v)
        pltpu.sync_copy(stage_v, d_sh.at[pl.ds(sid * _DPW, _DPW)])
        pltpu.sync_copy(idx_hbm.at[wid], idx_v)
        plsc.subcore_barrier()
        nch = _GPW // _CHUNK
        cpr = _N // _CHUNK
        copies = [
            pltpu.async_copy(d_sh.at[idx_v.at[j]],
                             rows_v.at[j // cpr, pl.ds((j % cpr) * _CHUNK,
                                                       _CHUNK)], sem)
            for j in range(nch)
        ]
        for c_ in copies:
            c_.wait()
        pltpu.sync_copy(rows_v, out.at[pl.ds(wid * (_N // _NW), _N // _NW)])

    return k(d, idx)


def _adj_index_map():
    ii = np.arange(_N, dtype=np.int32)[:, None]
    jj = np.arange(_N, dtype=np.int32)[None, :]
    lo = np.minimum(ii, jj)
    hi = np.maximum(ii, jj)
    start = lo * (_N - 1) - (lo * (lo - 1)) // 2
    k = start + hi - lo - 1
    k = np.where(ii == jj, _U + ii, k).astype(np.int32)
    return jnp.asarray(k.reshape(_NW, _GPW // _CHUNK, _CHUNK))


def kernel(x, edge_index, eps, W1, b1, g1, be1, W2, b2, g2, be2,
           Wmu, bmu, Wlv, blv, Wd1, bd1, Wd2, bd2):
    counts = _edge_counts(edge_index)
    d = _encode_decode(
        counts, x, W1, b1.reshape(1, -1), g1.reshape(1, -1), be1.reshape(1, -1),
        W2, b2.reshape(1, -1), g2.reshape(1, -1), be2.reshape(1, -1),
        Wmu, bmu.reshape(1, -1), Wlv, blv.reshape(1, -1),
        Wd1, bd1.reshape(1, -1), eps, Wd2, bd2)
    return _adj_gather(d, _adj_index_map())


# speedup vs baseline: 1.0654x; 1.0265x over previous
import functools

import jax
import jax.numpy as jnp
import numpy as np
from jax import lax
from jax.experimental import pallas as pl
from jax.experimental.pallas import tpu as pltpu
from jax.experimental.pallas import tpu_sc as plsc

_N = 512
_E = 16384
_U = _N * (_N - 1) // 2
_NUM_LOGITS = _U + _N
_NW = 32
_EPW = _E // _NW
_CHUNK = 128
_SLICE = _N * _N // 16
_GPW = _N * _N // _NW
_TILE = 12288
_DPW = _NUM_LOGITS // 16


def _sc_mesh():
    return plsc.VectorSubcoreMesh(core_axis_name="c", subcore_axis_name="s")


def _edge_counts(edge_index):
    @functools.partial(
        pl.kernel,
        out_type=jax.ShapeDtypeStruct((2, _N, _N), jnp.float32),
        mesh=_sc_mesh(),
        scratch_types=[
            pltpu.VMEM((_EPW,), jnp.int32),
            pltpu.VMEM((_EPW,), jnp.int32),
            pltpu.VMEM((_EPW // _CHUNK, _CHUNK), jnp.int32),
            pltpu.VMEM((_CHUNK,), jnp.float32),
            pltpu.VMEM((2048,), jnp.float32),
            pltpu.VMEM((_SLICE // _N, _N), jnp.float32),
            pltpu.VMEM_SHARED((_N * _N,), jnp.float32),
            pltpu.SemaphoreType.DMA,
            pltpu.SemaphoreType.DMA,
        ],
    )
    def k(edges, out, src_v, dst_v, idx_v, ones_v, zeros_v, rows_v, acc, sem,
          sem_ld):
        cid = lax.axis_index("c")
        sid = lax.axis_index("s")
        wid = cid * 16 + sid
        base = wid * _EPW
        ld_s = pltpu.async_copy(edges.at[0, pl.ds(base, _EPW)], src_v, sem_ld)
        ld_d = pltpu.async_copy(edges.at[1, pl.ds(base, _EPW)], dst_v, sem_ld)

        def _zero(i, carry):
            zeros_v[pl.ds(i * 16, 16)] = jnp.zeros((16,), jnp.float32)
            return carry

        lax.fori_loop(0, 128, _zero, 0)
        for t in range(_CHUNK // 16):
            ones_v[pl.ds(t * 16, 16)] = jnp.ones((16,), jnp.float32)
        zs = [
            pltpu.async_copy(zeros_v,
                             acc.at[pl.ds(sid * _SLICE + q * 2048, 2048)], sem)
            for q in range(_SLICE // 2048)
        ]
        ld_s.wait()
        ld_d.wait()
        for t in range(_EPW // 16):
            s = src_v[pl.ds(t * 16, 16)]
            d = dst_v[pl.ds(t * 16, 16)]
            idx_v[t * 16 // _CHUNK, pl.ds(t * 16 % _CHUNK, 16)] = d * _N + s
        for z in zs:
            z.wait()
        plsc.subcore_barrier()
        for c in range(_EPW // _CHUNK):
            pltpu.sync_copy(ones_v, acc.at[idx_v.at[c]], add=True)
        plsc.subcore_barrier()
        nrow = _SLICE // _N
        stages = [
            pltpu.async_copy(acc.at[pl.ds(sid * _SLICE + r * _N, _N)],
                             rows_v.at[r], sem)
            for r in range(nrow)
        ]
        for s_ in stages:
            s_.wait()
        pltpu.sync_copy(rows_v, out.at[cid, pl.ds(sid * nrow, nrow)])

    return k(edge_index)


def _encode_decode(counts, x, W1, b1, g1, be1, W2, b2, g2, be2,
                   Wmu, bmu, Wlv, blv, Wd1, bd1, eps, Wd2, bd2):
    def body(c_ref, x_ref, w1, b1r, g1r, be1r, w2, b2r, g2r, be2r,
             wmu, bmur, wlv, blvr, wd1, bd1r, eps_ref,
             wq0, wq1, wq2, wq3, b_ref, o_ref, a_scr):
        @pl.when(pl.program_id(0) == 0)
        def _():
            A = c_ref[0] + c_ref[1]
            deg = jnp.sum(A, axis=1, keepdims=True) + 1.0
            dinv = lax.rsqrt(deg)

            def gcn_bn_relu(h, W, b, g, be):
                gmat = jnp.dot(h, W, preferred_element_type=jnp.float32) * dinv
                m = (jnp.dot(A, gmat, preferred_element_type=jnp.float32)
                     + gmat) * dinv + b
                mu_ = jnp.sum(m, axis=0, keepdims=True) * (1.0 / _N)
                var = jnp.sum((m - mu_) ** 2, axis=0, keepdims=True) * (1.0 / _N)
                return jnp.maximum(
                    (m - mu_) * lax.rsqrt(var + 1e-5) * g + be, 0.0)

            h = gcn_bn_relu(x_ref[...], w1[...], b1r[...], g1r[...], be1r[...])
            h = gcn_bn_relu(h, w2[...], b2r[...], g2r[...], be2r[...])
            hg = jnp.sum(h, axis=0, keepdims=True) * (1.0 / _N)
            mu = (jnp.dot(hg, wmu[...], preferred_element_type=jnp.float32)
                  + bmur[...])
            logvar = (jnp.dot(hg, wlv[...], preferred_element_type=jnp.float32)
                      + blvr[...])
            z = mu + eps_ref[...] * jnp.exp(0.5 * logvar)
            a_scr[...] = jnp.maximum(
                jnp.dot(z, wd1[...], preferred_element_type=jnp.float32)
                + bd1r[...], 0.0)

        acc = jnp.broadcast_to(b_ref[...], (1, _TILE))
        for q, w in enumerate((wq0, wq1, wq2, wq3)):
            acc = acc + jnp.dot(a_scr[:, q * 64:(q + 1) * 64], w[...],
                                preferred_element_type=jnp.float32)
        o_ref[...] = jnp.tanh(acc).reshape(_TILE)

    enc_spec = lambda *shape: pl.BlockSpec(shape, lambda g: (0,) * len(shape))
    return pl.pallas_call(
        body,
        grid=(pl.cdiv(_NUM_LOGITS, _TILE),),
        in_specs=[
            enc_spec(2, _N, _N),
            enc_spec(*x.shape),
            enc_spec(*W1.shape), enc_spec(*b1.shape),
            enc_spec(*g1.shape), enc_spec(*be1.shape),
            enc_spec(*W2.shape), enc_spec(*b2.shape),
            enc_spec(*g2.shape), enc_spec(*be2.shape),
            enc_spec(*Wmu.shape), enc_spec(*bmu.shape),
            enc_spec(*Wlv.shape), enc_spec(*blv.shape),
            enc_spec(*Wd1.shape), enc_spec(*bd1.shape),
            enc_spec(*eps.shape),
            pl.BlockSpec((64, _TILE), lambda g: (0, g)),
            pl.BlockSpec((64, _TILE), lambda g: (1, g)),
            pl.BlockSpec((64, _TILE), lambda g: (2, g)),
            pl.BlockSpec((64, _TILE), lambda g: (3, g)),
            pl.BlockSpec((_TILE,), lambda g: (g,)),
        ],
        out_specs=pl.BlockSpec((_TILE,), lambda g: (g,)),
        out_shape=jax.ShapeDtypeStruct((_NUM_LOGITS,), jnp.float32),
        scratch_shapes=[pltpu.VMEM((1, 256), jnp.float32)],
        compiler_params=pltpu.CompilerParams(
            dimension_semantics=("arbitrary",)),
    )(counts, x, W1, b1, g1, be1, W2, b2, g2, be2,
      Wmu, bmu, Wlv, blv, Wd1, bd1, eps, Wd2, Wd2, Wd2, Wd2, bd2)


def _adj_gather(d, idx):
    @functools.partial(
        pl.kernel,
        out_type=jax.ShapeDtypeStruct((_N, _N), jnp.float32),
        mesh=_sc_mesh(),
        scratch_types=[
            pltpu.VMEM((_GPW // _CHUNK, _CHUNK), jnp.int32),
            pltpu.VMEM((_N // _NW, _N), jnp.float32),
            pltpu.VMEM((_DPW,), jnp.float32),
            pltpu.VMEM_SHARED((_NUM_LOGITS,), jnp.float32),
            pltpu.SemaphoreType.DMA,
            pltpu.SemaphoreType.DMA,
        ],
    )
    def k(d_hbm, idx_hbm, out, idx_v, rows_v, stage_v, d_sh, sem, sem_ld):
        cid = lax.axis_index("c")
        sid = lax.axis_index("s")
        wid = cid * 16 + sid
        st = pltpu.async_copy(d_hbm.at[pl.ds(sid * _DPW, _DPW)], stage_v,
                              sem_ld)
        ix = pltpu.async_copy(idx_hbm.at[wid], idx_v, sem_ld)
        st.wait()
        ix.wait()
        pltpu.sync_copy(stage_v, d_sh.at[pl.ds(sid * _DPW, _DPW)])
        plsc.subcore_barrier()
        nch = _GPW // _CHUNK
        cpr = _N // _CHUNK
        copies = [
            pltpu.async_copy(d_sh.at[idx_v.at[j]],
                             rows_v.at[j // cpr, pl.ds((j % cpr) * _CHUNK,
                                                       _CHUNK)], sem)
            for j in range(nch)
        ]
        for c_ in copies:
            c_.wait()
        pltpu.sync_copy(rows_v, out.at[pl.ds(wid * (_N // _NW), _N // _NW)])

    return k(d, idx)


def _adj_index_map():
    ii = np.arange(_N, dtype=np.int32)[:, None]
    jj = np.arange(_N, dtype=np.int32)[None, :]
    lo = np.minimum(ii, jj)
    hi = np.maximum(ii, jj)
    start = lo * (_N - 1) - (lo * (lo - 1)) // 2
    k = start + hi - lo - 1
    k = np.where(ii == jj, _U + ii, k).astype(np.int32)
    return jnp.asarray(k.reshape(_NW, _GPW // _CHUNK, _CHUNK))


def kernel(x, edge_index, eps, W1, b1, g1, be1, W2, b2, g2, be2,
           Wmu, bmu, Wlv, blv, Wd1, bd1, Wd2, bd2):
    counts = _edge_counts(edge_index)
    d = _encode_decode(
        counts, x, W1, b1.reshape(1, -1), g1.reshape(1, -1), be1.reshape(1, -1),
        W2, b2.reshape(1, -1), g2.reshape(1, -1), be2.reshape(1, -1),
        Wmu, bmu.reshape(1, -1), Wlv, blv.reshape(1, -1),
        Wd1, bd1.reshape(1, -1), eps, Wd2, bd2)
    return _adj_gather(d, _adj_index_map())
